# Initial kernel scaffold; baseline (speedup 1.0000x reference)
import jax
import jax.numpy as jnp
from jax.experimental import pallas as pl


def kernel(x, edge_index, W1, b1, W2, b2):
    raise NotImplementedError("write your pallas kernel here")


# speedup vs baseline: 4.8195x; 4.8195x over previous
import functools

import jax
import jax.numpy as jnp
from jax import lax
from jax.experimental import pallas as pl
from jax.experimental.pallas import tpu as pltpu
from jax.experimental.pallas import tpu_sc as plsc

N = 10000
E = 320000
NC = 2
NS = 16
NW = NC * NS
EPW = E // NW
CHUNK = 80
NCHUNK = EPW // CHUNK
NP = 10240
RPT = NP // NS

_SELU_ALPHA = 1.6732632423543772
_SELU_SCALE = 1.0507009873554805


def _make_prop(D):
    mesh = plsc.VectorSubcoreMesh(core_axis_name="c", subcore_axis_name="s")

    @functools.partial(
        pl.kernel,
        mesh=mesh,
        compiler_params=pltpu.CompilerParams(use_tc_tiling_on_sc=False),
        out_type=jax.ShapeDtypeStruct((NC, NP, D), jnp.float32),
        scratch_types=[
            pltpu.VMEM((CHUNK,), jnp.int32),
            pltpu.VMEM((CHUNK,), jnp.int32),
            pltpu.VMEM((CHUNK, D), jnp.float32),
            pltpu.VMEM_SHARED((NP, D), jnp.float32),
            pltpu.SemaphoreType.DMA,
        ],
    )
    def prop(h_hbm, src_hbm, dst_hbm, out_hbm, srcv, dstv, rows, acc, sem):
        cid = lax.axis_index("c")
        sid = lax.axis_index("s")
        wid = cid * NS + sid
        zeros16 = jnp.zeros((16,), jnp.float32)

        def zbody(r, carry):
            for c in range(D // 16):
                rows[r, pl.ds(c * 16, 16)] = zeros16
            return carry

        lax.fori_loop(0, CHUNK, zbody, 0)
        for k in range(RPT // CHUNK):
            pltpu.sync_copy(rows, acc.at[pl.ds(sid * RPT + k * CHUNK, CHUNK)])
        plsc.subcore_barrier()

        base = wid * EPW

        def ebody(i, carry):
            off = base + i * CHUNK
            pltpu.sync_copy(src_hbm.at[pl.ds(off, CHUNK)], srcv)
            pltpu.sync_copy(dst_hbm.at[pl.ds(off, CHUNK)], dstv)
            pltpu.async_copy(h_hbm.at[srcv], rows, sem).wait()
            pltpu.sync_copy(rows, acc.at[dstv], add=True)
            return carry

        lax.fori_loop(0, NCHUNK, ebody, 0)
        plsc.subcore_barrier()
        pltpu.sync_copy(
            acc.at[pl.ds(sid * RPT, RPT)],
            out_hbm.at[cid, pl.ds(sid * RPT, RPT)],
        )

    return prop


_prop128 = _make_prop(128)
_prop64 = _make_prop(64)

_BR = 1000
_NB = N // _BR


def _combine(p):
    D = p.shape[2]

    def body(a_ref, b_ref, o_ref):
        o_ref[...] = a_ref[0] + b_ref[0]

    return pl.pallas_call(
        body,
        grid=(_NB,),
        in_specs=[
            pl.BlockSpec((1, _BR, D), lambda i: (0, i, 0)),
            pl.BlockSpec((1, _BR, D), lambda i: (1, i, 0)),
        ],
        out_specs=pl.BlockSpec((_BR, D), lambda i: (i, 0)),
        out_shape=jax.ShapeDtypeStruct((N, D), jnp.float32),
    )(p, p)


def _mlp(p2, W1, b1, W2):
    D = p2.shape[2]
    DO = W2.shape[1]

    def body(a_ref, b_ref, w1_ref, b1_ref, w2_ref, o_ref):
        h = a_ref[0] + b_ref[0]
        h = jnp.dot(h, w1_ref[...], preferred_element_type=jnp.float32)
        h = h + b1_ref[...]
        h = _SELU_SCALE * jnp.where(h > 0, h, _SELU_ALPHA * (jnp.exp(h) - 1.0))
        o_ref[...] = jnp.dot(h, w2_ref[...], preferred_element_type=jnp.float32)

    return pl.pallas_call(
        body,
        grid=(_NB,),
        in_specs=[
            pl.BlockSpec((1, _BR, D), lambda i: (0, i, 0)),
            pl.BlockSpec((1, _BR, D), lambda i: (1, i, 0)),
            pl.BlockSpec(W1.shape, lambda i: (0, 0)),
            pl.BlockSpec((1, D), lambda i: (0, 0)),
            pl.BlockSpec(W2.shape, lambda i: (0, 0)),
        ],
        out_specs=pl.BlockSpec((_BR, DO), lambda i: (i, 0)),
        out_shape=jax.ShapeDtypeStruct((N, DO), jnp.float32),
    )(p2, p2, W1, b1.reshape(1, D), W2)


def _final(p3, g, b2):
    DO = g.shape[1]

    def body(a_ref, b_ref, g_ref, b2_ref, o_ref):
        s = a_ref[0] + b_ref[0] + g_ref[...] + b2_ref[...]
        s = s - jnp.max(s, axis=1, keepdims=True)
        o_ref[...] = s - jnp.log(jnp.sum(jnp.exp(s), axis=1, keepdims=True))

    return pl.pallas_call(
        body,
        grid=(_NB,),
        in_specs=[
            pl.BlockSpec((1, _BR, DO), lambda i: (0, i, 0)),
            pl.BlockSpec((1, _BR, DO), lambda i: (1, i, 0)),
            pl.BlockSpec((_BR, DO), lambda i: (i, 0)),
            pl.BlockSpec((1, DO), lambda i: (0, 0)),
        ],
        out_specs=pl.BlockSpec((_BR, DO), lambda i: (i, 0)),
        out_shape=jax.ShapeDtypeStruct((N, DO), jnp.float32),
    )(p3, p3, g, b2.reshape(1, DO))


def kernel(x, edge_index, W1, b1, W2, b2):
    src = edge_index[0]
    dst = edge_index[1]
    p1 = _prop128(x, src, dst)
    h1 = _combine(p1)
    p2 = _prop128(h1, src, dst)
    g = _mlp(p2, W1, b1, W2)
    p3 = _prop64(g, src, dst)
    return _final(p3, g, b2)


# speedup vs baseline: 11.4889x; 2.3838x over previous
import functools

import jax
import jax.numpy as jnp
from jax import lax
from jax.experimental import pallas as pl
from jax.experimental.pallas import tpu as pltpu
from jax.experimental.pallas import tpu_sc as plsc

N = 10000
E = 320000
NC = 2
NS = 16
NW = NC * NS
EPW = E // NW
CHUNK = 80
NCHUNK = EPW // CHUNK
NP = 10240
RPT = NP // NS

_SELU_ALPHA = 1.6732632423543772
_SELU_SCALE = 1.0507009873554805


def _make_prop(D):
    mesh = plsc.VectorSubcoreMesh(core_axis_name="c", subcore_axis_name="s")

    @functools.partial(
        pl.kernel,
        mesh=mesh,
        compiler_params=pltpu.CompilerParams(use_tc_tiling_on_sc=False),
        out_type=jax.ShapeDtypeStruct((NC, NP, D), jnp.float32),
        scratch_types=[
            pltpu.VMEM((NCHUNK, CHUNK), jnp.int32),
            pltpu.VMEM((NCHUNK, CHUNK), jnp.int32),
            pltpu.VMEM((CHUNK, D), jnp.float32),
            pltpu.VMEM((CHUNK, D), jnp.float32),
            pltpu.VMEM_SHARED((NP, D), jnp.float32),
            pltpu.SemaphoreType.DMA,
            pltpu.SemaphoreType.DMA,
        ],
    )
    def prop(h_hbm, src_hbm, dst_hbm, out_hbm, srcb, dstb, rows0, rows1,
             acc, sem0, sem1):
        cid = lax.axis_index("c")
        sid = lax.axis_index("s")
        wid = cid * NS + sid
        pltpu.sync_copy(src_hbm.at[wid], srcb)
        pltpu.sync_copy(dst_hbm.at[wid], dstb)
        zeros16 = jnp.zeros((16,), jnp.float32)

        def zbody(r, carry):
            for c in range(D // 16):
                rows0[r, pl.ds(c * 16, 16)] = zeros16
            return carry

        lax.fori_loop(0, CHUNK, zbody, 0)
        for k in range(RPT // CHUNK):
            pltpu.sync_copy(rows0, acc.at[pl.ds(sid * RPT + k * CHUNK, CHUNK)])
        plsc.subcore_barrier()

        def gather(i, rbuf, sem):
            return pltpu.make_async_copy(h_hbm.at[srcb.at[i]], rbuf, sem)

        def scat(i, rbuf):
            pltpu.sync_copy(rbuf, acc.at[dstb.at[i]], add=True)

        gather(0, rows0, sem0).start()

        def pair(j, carry):
            i0 = 2 * j
            gather(i0 + 1, rows1, sem1).start()
            gather(i0, rows0, sem0).wait()
            scat(i0, rows0)
            gather(i0 + 2, rows0, sem0).start()
            gather(i0 + 1, rows1, sem1).wait()
            scat(i0 + 1, rows1)
            return carry

        lax.fori_loop(0, (NCHUNK - 1) // 2, pair, 0)
        gather(NCHUNK - 1, rows0, sem0).wait()
        scat(NCHUNK - 1, rows0)
        plsc.subcore_barrier()
        pltpu.sync_copy(
            acc.at[pl.ds(sid * RPT, RPT)],
            out_hbm.at[cid, pl.ds(sid * RPT, RPT)],
        )

    return prop


_prop128 = _make_prop(128)
_prop64 = _make_prop(64)

_BR = 1000
_NB = N // _BR


def _combine(p):
    D = p.shape[2]

    def body(a_ref, b_ref, o_ref):
        o_ref[...] = a_ref[0] + b_ref[0]

    return pl.pallas_call(
        body,
        grid=(_NB,),
        in_specs=[
            pl.BlockSpec((1, _BR, D), lambda i: (0, i, 0)),
            pl.BlockSpec((1, _BR, D), lambda i: (1, i, 0)),
        ],
        out_specs=pl.BlockSpec((_BR, D), lambda i: (i, 0)),
        out_shape=jax.ShapeDtypeStruct((N, D), jnp.float32),
    )(p, p)


def _mlp(p2, W1, b1, W2):
    D = p2.shape[2]
    DO = W2.shape[1]

    def body(a_ref, b_ref, w1_ref, b1_ref, w2_ref, o_ref):
        h = a_ref[0] + b_ref[0]
        h = jnp.dot(h, w1_ref[...], preferred_element_type=jnp.float32)
        h = h + b1_ref[...]
        h = _SELU_SCALE * jnp.where(h > 0, h, _SELU_ALPHA * (jnp.exp(h) - 1.0))
        o_ref[...] = jnp.dot(h, w2_ref[...], preferred_element_type=jnp.float32)

    return pl.pallas_call(
        body,
        grid=(_NB,),
        in_specs=[
            pl.BlockSpec((1, _BR, D), lambda i: (0, i, 0)),
            pl.BlockSpec((1, _BR, D), lambda i: (1, i, 0)),
            pl.BlockSpec(W1.shape, lambda i: (0, 0)),
            pl.BlockSpec((1, D), lambda i: (0, 0)),
            pl.BlockSpec(W2.shape, lambda i: (0, 0)),
        ],
        out_specs=pl.BlockSpec((_BR, DO), lambda i: (i, 0)),
        out_shape=jax.ShapeDtypeStruct((N, DO), jnp.float32),
    )(p2, p2, W1, b1.reshape(1, D), W2)


def _final(p3, g, b2):
    DO = g.shape[1]

    def body(a_ref, b_ref, g_ref, b2_ref, o_ref):
        s = a_ref[0] + b_ref[0] + g_ref[...] + b2_ref[...]
        s = s - jnp.max(s, axis=1, keepdims=True)
        o_ref[...] = s - jnp.log(jnp.sum(jnp.exp(s), axis=1, keepdims=True))

    return pl.pallas_call(
        body,
        grid=(_NB,),
        in_specs=[
            pl.BlockSpec((1, _BR, DO), lambda i: (0, i, 0)),
            pl.BlockSpec((1, _BR, DO), lambda i: (1, i, 0)),
            pl.BlockSpec((_BR, DO), lambda i: (i, 0)),
            pl.BlockSpec((1, DO), lambda i: (0, 0)),
        ],
        out_specs=pl.BlockSpec((_BR, DO), lambda i: (i, 0)),
        out_shape=jax.ShapeDtypeStruct((N, DO), jnp.float32),
    )(p3, p3, g, b2.reshape(1, DO))


def kernel(x, edge_index, W1, b1, W2, b2):
    src = edge_index[0].reshape(NW, NCHUNK, CHUNK)
    dst = edge_index[1].reshape(NW, NCHUNK, CHUNK)
    p1 = _prop128(x, src, dst)
    h1 = _combine(p1)
    p2 = _prop128(h1, src, dst)
    g = _mlp(p2, W1, b1, W2)
    p3 = _prop64(g, src, dst)
    return _final(p3, g, b2)


# speedup vs baseline: 12.0859x; 1.0520x over previous
import functools

import jax
import jax.numpy as jnp
from jax import lax
from jax.experimental import pallas as pl
from jax.experimental.pallas import tpu as pltpu
from jax.experimental.pallas import tpu_sc as plsc

N = 10000
E = 320000
NC = 2
NS = 16
NW = NC * NS
EPW = E // NW
CHUNK = 40
NCHUNK = EPW // CHUNK
NBUF = 5
NROUND = NCHUNK // NBUF
NP = 10240
RPT = NP // NS

_SELU_ALPHA = 1.6732632423543772
_SELU_SCALE = 1.0507009873554805


def _make_prop(D):
    mesh = plsc.VectorSubcoreMesh(core_axis_name="c", subcore_axis_name="s")

    @functools.partial(
        pl.kernel,
        mesh=mesh,
        compiler_params=pltpu.CompilerParams(use_tc_tiling_on_sc=False),
        out_type=jax.ShapeDtypeStruct((NC, NP, D), jnp.float32),
        scratch_types=[
            pltpu.VMEM((NCHUNK, CHUNK), jnp.int32),
            pltpu.VMEM((NCHUNK, CHUNK), jnp.int32),
            [pltpu.VMEM((CHUNK, D), jnp.float32) for _ in range(NBUF)],
            pltpu.VMEM_SHARED((NP, D), jnp.float32),
            [pltpu.SemaphoreType.DMA for _ in range(NBUF)],
            [pltpu.SemaphoreType.DMA for _ in range(NBUF)],
        ],
    )
    def prop(h_hbm, src_hbm, dst_hbm, out_hbm, srcb, dstb, rows, acc,
             gsem, ssem):
        cid = lax.axis_index("c")
        sid = lax.axis_index("s")
        wid = cid * NS + sid
        pltpu.sync_copy(src_hbm.at[wid], srcb)
        pltpu.sync_copy(dst_hbm.at[wid], dstb)
        zeros16 = jnp.zeros((16,), jnp.float32)

        def zbody(r, carry):
            for c in range(D // 16):
                rows[0][r, pl.ds(c * 16, 16)] = zeros16
            return carry

        lax.fori_loop(0, CHUNK, zbody, 0)
        for k in range(RPT // CHUNK):
            pltpu.sync_copy(rows[0], acc.at[pl.ds(sid * RPT + k * CHUNK, CHUNK)])
        plsc.subcore_barrier()

        def gather_start(i, b):
            pltpu.async_copy(h_hbm.at[srcb.at[i]], rows[b], gsem[b])

        def gather_wait(i, b):
            pltpu.make_async_copy(h_hbm.at[srcb.at[i]], rows[b], gsem[b]).wait()

        def scat_start(i, b):
            pltpu.async_copy(rows[b], acc.at[dstb.at[i]], ssem[b], add=True)

        def scat_wait(i, b):
            pltpu.make_async_copy(rows[b], acc.at[dstb.at[i]], ssem[b]).wait()

        for b in range(NBUF):
            gather_start(b, b)

        def round_body(r, carry):
            i = r * NBUF
            for b in range(NBUF):
                gather_wait(i + b, b)
                scat_start(i + b, b)
            for b in range(NBUF):
                scat_wait(i + b, b)
                gather_start(i + NBUF + b, b)
            return carry

        lax.fori_loop(0, NROUND - 1, round_body, 0)
        i = (NROUND - 1) * NBUF
        for b in range(NBUF):
            gather_wait(i + b, b)
            scat_start(i + b, b)
        for b in range(NBUF):
            scat_wait(i + b, b)
        plsc.subcore_barrier()
        pltpu.sync_copy(
            acc.at[pl.ds(sid * RPT, RPT)],
            out_hbm.at[cid, pl.ds(sid * RPT, RPT)],
        )

    return prop


_prop128 = _make_prop(128)
_prop64 = _make_prop(64)

_BR = 1000
_NB = N // _BR


def _combine(p):
    D = p.shape[2]

    def body(a_ref, b_ref, o_ref):
        o_ref[...] = a_ref[0] + b_ref[0]

    return pl.pallas_call(
        body,
        grid=(_NB,),
        in_specs=[
            pl.BlockSpec((1, _BR, D), lambda i: (0, i, 0)),
            pl.BlockSpec((1, _BR, D), lambda i: (1, i, 0)),
        ],
        out_specs=pl.BlockSpec((_BR, D), lambda i: (i, 0)),
        out_shape=jax.ShapeDtypeStruct((N, D), jnp.float32),
    )(p, p)


def _mlp(p2, W1, b1, W2):
    D = p2.shape[2]
    DO = W2.shape[1]

    def body(a_ref, b_ref, w1_ref, b1_ref, w2_ref, o_ref):
        h = a_ref[0] + b_ref[0]
        h = jnp.dot(h, w1_ref[...], preferred_element_type=jnp.float32)
        h = h + b1_ref[...]
        h = _SELU_SCALE * jnp.where(h > 0, h, _SELU_ALPHA * (jnp.exp(h) - 1.0))
        o_ref[...] = jnp.dot(h, w2_ref[...], preferred_element_type=jnp.float32)

    return pl.pallas_call(
        body,
        grid=(_NB,),
        in_specs=[
            pl.BlockSpec((1, _BR, D), lambda i: (0, i, 0)),
            pl.BlockSpec((1, _BR, D), lambda i: (1, i, 0)),
            pl.BlockSpec(W1.shape, lambda i: (0, 0)),
            pl.BlockSpec((1, D), lambda i: (0, 0)),
            pl.BlockSpec(W2.shape, lambda i: (0, 0)),
        ],
        out_specs=pl.BlockSpec((_BR, DO), lambda i: (i, 0)),
        out_shape=jax.ShapeDtypeStruct((N, DO), jnp.float32),
    )(p2, p2, W1, b1.reshape(1, D), W2)


def _final(p3, g, b2):
    DO = g.shape[1]

    def body(a_ref, b_ref, g_ref, b2_ref, o_ref):
        s = a_ref[0] + b_ref[0] + g_ref[...] + b2_ref[...]
        s = s - jnp.max(s, axis=1, keepdims=True)
        o_ref[...] = s - jnp.log(jnp.sum(jnp.exp(s), axis=1, keepdims=True))

    return pl.pallas_call(
        body,
        grid=(_NB,),
        in_specs=[
            pl.BlockSpec((1, _BR, DO), lambda i: (0, i, 0)),
            pl.BlockSpec((1, _BR, DO), lambda i: (1, i, 0)),
            pl.BlockSpec((_BR, DO), lambda i: (i, 0)),
            pl.BlockSpec((1, DO), lambda i: (0, 0)),
        ],
        out_specs=pl.BlockSpec((_BR, DO), lambda i: (i, 0)),
        out_shape=jax.ShapeDtypeStruct((N, DO), jnp.float32),
    )(p3, p3, g, b2.reshape(1, DO))


def kernel(x, edge_index, W1, b1, W2, b2):
    src = edge_index[0].reshape(NW, NCHUNK, CHUNK)
    dst = edge_index[1].reshape(NW, NCHUNK, CHUNK)
    p1 = _prop128(x, src, dst)
    h1 = _combine(p1)
    p2 = _prop128(h1, src, dst)
    g = _mlp(p2, W1, b1, W2)
    p3 = _prop64(g, src, dst)
    return _final(p3, g, b2)


# speedup vs baseline: 14.0975x; 1.1664x over previous
import functools

import jax
import jax.numpy as jnp
from jax import lax
from jax.experimental import pallas as pl
from jax.experimental.pallas import tpu as pltpu
from jax.experimental.pallas import tpu_sc as plsc

N = 10000
E = 320000
NC = 2
NS = 16
NW = NC * NS
EPW = E // NW
CHUNK = 80
NCHUNK = EPW // CHUNK
NBUF = 5
NROUND = NCHUNK // NBUF
NP = 10240
RPT = NP // NS

_SELU_ALPHA = 1.6732632423543772
_SELU_SCALE = 1.0507009873554805


def _make_prop(D, dtype):
    mesh = plsc.VectorSubcoreMesh(core_axis_name="c", subcore_axis_name="s")
    lanes = 32 if dtype == jnp.bfloat16 else 16

    @functools.partial(
        pl.kernel,
        mesh=mesh,
        compiler_params=pltpu.CompilerParams(use_tc_tiling_on_sc=False),
        out_type=jax.ShapeDtypeStruct((NC, NP, D), dtype),
        scratch_types=[
            pltpu.VMEM((NCHUNK, CHUNK), jnp.int32),
            pltpu.VMEM((NCHUNK, CHUNK), jnp.int32),
            [pltpu.VMEM((CHUNK, D), dtype) for _ in range(NBUF)],
            pltpu.VMEM_SHARED((NP, D), dtype),
            [pltpu.SemaphoreType.DMA for _ in range(NBUF)],
            [pltpu.SemaphoreType.DMA for _ in range(NBUF)],
        ],
    )
    def prop(h_hbm, src_hbm, dst_hbm, out_hbm, srcb, dstb, rows, acc,
             gsem, ssem):
        cid = lax.axis_index("c")
        sid = lax.axis_index("s")
        wid = cid * NS + sid
        pltpu.sync_copy(src_hbm.at[wid], srcb)
        pltpu.sync_copy(dst_hbm.at[wid], dstb)
        zvec = jnp.zeros((lanes,), dtype)

        def zbody(r, carry):
            for c in range(D // lanes):
                rows[0][r, pl.ds(c * lanes, lanes)] = zvec
            return carry

        lax.fori_loop(0, CHUNK, zbody, 0)
        for k in range(RPT // CHUNK):
            pltpu.sync_copy(rows[0], acc.at[pl.ds(sid * RPT + k * CHUNK, CHUNK)])
        plsc.subcore_barrier()

        def gather_start(i, b):
            pltpu.async_copy(h_hbm.at[srcb.at[i]], rows[b], gsem[b])

        def gather_wait(i, b):
            pltpu.make_async_copy(h_hbm.at[srcb.at[i]], rows[b], gsem[b]).wait()

        def scat_start(i, b):
            pltpu.async_copy(rows[b], acc.at[dstb.at[i]], ssem[b], add=True)

        def scat_wait(i, b):
            pltpu.make_async_copy(rows[b], acc.at[dstb.at[i]], ssem[b]).wait()

        for b in range(NBUF):
            gather_start(b, b)

        def round_body(r, carry):
            i = r * NBUF
            for b in range(NBUF):
                gather_wait(i + b, b)
                scat_start(i + b, b)
            for b in range(NBUF):
                scat_wait(i + b, b)
                gather_start(i + NBUF + b, b)
            return carry

        lax.fori_loop(0, NROUND - 1, round_body, 0)
        i = (NROUND - 1) * NBUF
        for b in range(NBUF):
            gather_wait(i + b, b)
            scat_start(i + b, b)
        for b in range(NBUF):
            scat_wait(i + b, b)
        plsc.subcore_barrier()
        pltpu.sync_copy(
            acc.at[pl.ds(sid * RPT, RPT)],
            out_hbm.at[cid, pl.ds(sid * RPT, RPT)],
        )

    return prop


_prop128 = _make_prop(128, jnp.bfloat16)
_prop64 = _make_prop(64, jnp.float32)

_BR = 1000
_NB = N // _BR


def _combine(p):
    D = p.shape[2]

    def body(a_ref, b_ref, o_ref):
        s = a_ref[0].astype(jnp.float32) + b_ref[0].astype(jnp.float32)
        o_ref[...] = s.astype(jnp.bfloat16)

    return pl.pallas_call(
        body,
        grid=(_NB,),
        in_specs=[
            pl.BlockSpec((1, _BR, D), lambda i: (0, i, 0)),
            pl.BlockSpec((1, _BR, D), lambda i: (1, i, 0)),
        ],
        out_specs=pl.BlockSpec((_BR, D), lambda i: (i, 0)),
        out_shape=jax.ShapeDtypeStruct((N, D), jnp.bfloat16),
    )(p, p)


def _mlp(p2, W1, b1, W2):
    D = p2.shape[2]
    DO = W2.shape[1]

    def body(a_ref, b_ref, w1_ref, b1_ref, w2_ref, o_ref):
        h = a_ref[0].astype(jnp.float32) + b_ref[0].astype(jnp.float32)
        h = jnp.dot(h, w1_ref[...], preferred_element_type=jnp.float32)
        h = h + b1_ref[...]
        h = _SELU_SCALE * jnp.where(h > 0, h, _SELU_ALPHA * (jnp.exp(h) - 1.0))
        o_ref[...] = jnp.dot(h, w2_ref[...], preferred_element_type=jnp.float32)

    return pl.pallas_call(
        body,
        grid=(_NB,),
        in_specs=[
            pl.BlockSpec((1, _BR, D), lambda i: (0, i, 0)),
            pl.BlockSpec((1, _BR, D), lambda i: (1, i, 0)),
            pl.BlockSpec(W1.shape, lambda i: (0, 0)),
            pl.BlockSpec((1, D), lambda i: (0, 0)),
            pl.BlockSpec(W2.shape, lambda i: (0, 0)),
        ],
        out_specs=pl.BlockSpec((_BR, DO), lambda i: (i, 0)),
        out_shape=jax.ShapeDtypeStruct((N, DO), jnp.float32),
    )(p2, p2, W1, b1.reshape(1, D), W2)


def _final(p3, g, b2):
    DO = g.shape[1]

    def body(a_ref, b_ref, g_ref, b2_ref, o_ref):
        s = a_ref[0] + b_ref[0] + g_ref[...] + b2_ref[...]
        s = s - jnp.max(s, axis=1, keepdims=True)
        o_ref[...] = s - jnp.log(jnp.sum(jnp.exp(s), axis=1, keepdims=True))

    return pl.pallas_call(
        body,
        grid=(_NB,),
        in_specs=[
            pl.BlockSpec((1, _BR, DO), lambda i: (0, i, 0)),
            pl.BlockSpec((1, _BR, DO), lambda i: (1, i, 0)),
            pl.BlockSpec((_BR, DO), lambda i: (i, 0)),
            pl.BlockSpec((1, DO), lambda i: (0, 0)),
        ],
        out_specs=pl.BlockSpec((_BR, DO), lambda i: (i, 0)),
        out_shape=jax.ShapeDtypeStruct((N, DO), jnp.float32),
    )(p3, p3, g, b2.reshape(1, DO))


def kernel(x, edge_index, W1, b1, W2, b2):
    src = edge_index[0].reshape(NW, NCHUNK, CHUNK)
    dst = edge_index[1].reshape(NW, NCHUNK, CHUNK)
    p1 = _prop128(x.astype(jnp.bfloat16), src, dst)
    h1 = _combine(p1)
    p2 = _prop128(h1, src, dst)
    g = _mlp(p2, W1, b1, W2)
    p3 = _prop64(g, src, dst)
    return _final(p3, g, b2)
